# baseline (device time: 37152 ns/iter reference)
import jax
import jax.numpy as jnp
from jax import lax
from jax.experimental import pallas as pl
from jax.experimental.pallas import tpu as pltpu

N_DEV = 4
B = 512
H = 512
HALF = 128
N_LAYERS = 3
N_SEMS = N_LAYERS * 6

F32 = jnp.float32
BF16 = jnp.bfloat16


def kernel(x, Win0, Wout0, Win1, Wout1, Win2, Wout2):
    b, d_shard = x.shape

    def body(x_ref, win0_ref, wout0_ref, win1_ref, wout1_ref,
             win2_ref, wout2_ref, out_ref,
             h_ref, s1s, s1r, s2s, s2r, s3s, s3r,
             send_sems, recv_sems):
        my_pos = lax.axis_index("i")
        pA = my_pos ^ 1
        pB = 3 - my_pos
        k1T = (my_pos ^ (my_pos >> 1)) & 1
        k1U = my_pos >> 1

        tK_e = k1T * HALF
        tS_e = (1 - k1T) * HALF
        uK_e = 2 * HALF + k1U * HALF
        uS_e = 2 * HALF + (1 - k1U) * HALF

        def rows(layer):
            if layer % 2 == 0:
                return tK_e, tS_e, uK_e, uS_e
            return tS_e, tK_e, uS_e, uK_e

        barrier_sem = pltpu.get_barrier_semaphore()
        for nbr in (pA, pB):
            pl.semaphore_signal(
                barrier_sem, inc=1,
                device_id=(nbr,), device_id_type=pl.DeviceIdType.MESH,
            )
        pl.semaphore_wait(barrier_sem, 2)

        def exch(src_ref, dst_ref, sem_idx, partner):
            return pltpu.make_async_remote_copy(
                src_ref=src_ref,
                dst_ref=dst_ref,
                send_sem=send_sems.at[sem_idx],
                recv_sem=recv_sems.at[sem_idx],
                device_id=(partner,),
                device_id_type=pl.DeviceIdType.MESH,
            )

        def start_s1(layer, ht_bf16, hu_bf16):
            s1s[0] = ht_bf16
            s1s[1] = hu_bf16
            rT = exch(s1s.at[0], s1r.at[0], layer * 6 + 0, pA)
            rU = exch(s1s.at[1], s1r.at[1], layer * 6 + 1, pB)
            rT.start()
            rU.start()
            return rT, rU

        win_refs = [win0_ref, win1_ref, win2_ref]
        wout_refs = [wout0_ref, wout1_ref, wout2_ref]

        def dot(a_bf16, w_bf16):
            return jnp.dot(a_bf16, w_bf16, preferred_element_type=F32)

        tK, tS, uK, uS = rows(0)
        win_bf = win0_ref[:, :].astype(BF16)
        s1T, s1U = start_s1(
            0,
            dot(x_ref[pl.ds(tS, HALF), :].astype(BF16), win_bf).astype(BF16),
            dot(x_ref[pl.ds(uS, HALF), :].astype(BF16), win_bf).astype(BF16),
        )
        h_ref[pl.ds(tK, HALF), :] = dot(
            x_ref[pl.ds(tK, HALF), :].astype(BF16), win_bf)
        h_ref[pl.ds(uK, HALF), :] = dot(
            x_ref[pl.ds(uK, HALF), :].astype(BF16), win_bf)

        for layer in range(N_LAYERS):
            tK, tS, uK, uS = rows(layer)
            wout_bf = wout_refs[layer][:, :].astype(BF16)
            base = layer * 6

            s1T.wait()
            s1U.wait()
            h2_t = h_ref[pl.ds(tK, HALF), :] + s1r[0].astype(F32)
            h2_u = h_ref[pl.ds(uK, HALF), :] + s1r[1].astype(F32)

            s2s[0] = h2_t.astype(BF16)
            s2s[1] = h2_u.astype(BF16)
            rT = exch(s2s.at[0], s2r.at[0], base + 2, pB)
            rU = exch(s2s.at[1], s2r.at[1], base + 3, pA)
            rT.start()
            rU.start()
            rT.wait()
            rU.wait()
            relu_t = jnp.maximum(h2_t + s2r[0].astype(F32), 0.0)
            relu_u = jnp.maximum(h2_u + s2r[1].astype(F32), 0.0)

            relu_t_bf = relu_t.astype(BF16)
            relu_u_bf = relu_u.astype(BF16)
            s3s[0] = relu_t_bf
            s3s[1] = relu_u_bf
            rT = exch(s3s.at[0], s3r.at[0], base + 4, pA)
            rU = exch(s3s.at[1], s3r.at[1], base + 5, pB)
            rT.start()
            rU.start()

            xk_t = dot(relu_t_bf, wout_bf)
            xk_u = dot(relu_u_bf, wout_bf)
            if layer < N_LAYERS - 1:
                win_n_bf = win_refs[layer + 1][:, :].astype(BF16)
                s1T, s1U = start_s1(
                    layer + 1,
                    dot(xk_t.astype(BF16), win_n_bf).astype(BF16),
                    dot(xk_u.astype(BF16), win_n_bf).astype(BF16),
                )

            rT.wait()
            rU.wait()
            xs_t = dot(s3r[0], wout_bf)
            xs_u = dot(s3r[1], wout_bf)

            if layer < N_LAYERS - 1:
                h_ref[pl.ds(tS, HALF), :] = dot(xs_t.astype(BF16), win_n_bf)
                h_ref[pl.ds(uS, HALF), :] = dot(xs_u.astype(BF16), win_n_bf)
            else:
                out_ref[pl.ds(tK, HALF), :] = xk_t
                out_ref[pl.ds(uK, HALF), :] = xk_u
                out_ref[pl.ds(tS, HALF), :] = xs_t
                out_ref[pl.ds(uS, HALF), :] = xs_u

    return pl.pallas_call(
        body,
        out_shape=jax.ShapeDtypeStruct((b, d_shard), F32),
        in_specs=[pl.BlockSpec(memory_space=pltpu.VMEM)] * 7,
        out_specs=pl.BlockSpec(memory_space=pltpu.VMEM),
        scratch_shapes=[
            pltpu.VMEM((B, H), F32),
            pltpu.VMEM((2, HALF, H), BF16),
            pltpu.VMEM((2, HALF, H), BF16),
            pltpu.VMEM((2, HALF, H), BF16),
            pltpu.VMEM((2, HALF, H), BF16),
            pltpu.VMEM((2, HALF, H), BF16),
            pltpu.VMEM((2, HALF, H), BF16),
            pltpu.SemaphoreType.DMA((N_SEMS,)),
            pltpu.SemaphoreType.DMA((N_SEMS,)),
        ],
        compiler_params=pltpu.CompilerParams(collective_id=0),
    )(x, Win0, Wout0, Win1, Wout1, Win2, Wout2)
